# baseline (speedup 1.0000x reference)
import functools

import jax
import jax.numpy as jnp
from jax import lax
from jax.experimental import pallas as pl
from jax.experimental.pallas import tpu as pltpu
from jax.experimental.pallas import tpu_sc as plsc

D = 128
HIST = 200
BATCH = 4096
SEG_STRIDE = 201
PAD0 = 2 * SEG_STRIDE
PAD1 = PAD0 + 32
CROWS = 512
XW = HIST // 2
BW = BATCH // 16
L = 16
G = 128


def _build_tbl_body(pe_ref, sw_ref, out_ref):
    pe = pe_ref[0:SEG_STRIDE]
    sw0 = sw_ref[0:1]
    sw1 = sw_ref[1:2]
    out_ref[0:SEG_STRIDE] = pe + sw0
    out_ref[SEG_STRIDE:PAD0] = pe + sw1
    zeros = jnp.zeros((CROWS - PAD0, D), jnp.float32)
    out_ref[PAD0:PAD1] = zeros[: PAD1 - PAD0] + sw0
    out_ref[PAD1:CROWS] = zeros[: CROWS - PAD1] + sw1


def _build_combined_table(pe_table, seg_weight):
    return pl.pallas_call(
        _build_tbl_body,
        out_shape=jax.ShapeDtypeStruct((CROWS, D), jnp.float32),
    )(pe_table[0:SEG_STRIDE], seg_weight)


def _sc_body(off_hbm, len_hbm, c_hbm, out_hbm,
             off_v, len_v, tbl_sh,
             idx00, idx01, idx10, idx11, rows0, rows1,
             gsem00, gsem01, gsem10, gsem11, ssem0, ssem1):
    cid = lax.axis_index("c")
    sid = lax.axis_index("s")
    wid = sid * 2 + cid
    xbase = cid * XW
    b0 = sid * BW

    idx = ((idx00, idx01), (idx10, idx11))
    rows = (rows0, rows1)
    gsem = ((gsem00, gsem01), (gsem10, gsem11))
    ssem = (ssem0, ssem1)

    tchunk = CROWS // 16
    tsl = pl.ds(sid * tchunk, tchunk)
    pltpu.sync_copy(c_hbm.at[tsl], rows0.at[pl.ds(0, tchunk)])
    pltpu.sync_copy(rows0.at[pl.ds(0, tchunk)], tbl_sh.at[tsl])
    plsc.subcore_barrier()

    pltpu.sync_copy(off_hbm.at[pl.ds(b0, BW)], off_v)
    pltpu.sync_copy(len_hbm.at[pl.ds(b0, BW)], len_v)

    def compute_idx(m, j):
        x = xbase + m
        for k in range(2):
            for s in range(G // L):
                sl = pl.ds(k * G + s * L, L)
                pos = off_v[sl]
                ln = len_v[sl]
                lt = x < pos
                rel = jnp.where(lt, pos - x, x + 1 - pos)
                live = rel + jnp.where(lt, 0, SEG_STRIDE)
                pad = PAD0 + wid + jnp.where(lt, 0, 32)
                idx[j][k][pl.ds(s * L, L)] = jnp.where(x < ln, live, pad)

    def gathers(j):
        return [pltpu.make_async_copy(tbl_sh.at[idx[j][k]],
                                      rows[j].at[pl.ds(k * G, G)],
                                      gsem[j][k])
                for k in range(2)]

    def store(j, m):
        dst = out_hbm.at[pl.ds((xbase + m) * BATCH + b0, BW)]
        return pltpu.make_async_copy(rows[j], dst, ssem[j])

    compute_idx(0, 0)
    for g in gathers(0):
        g.start()
    compute_idx(1, 1)
    for g in gathers(1):
        g.start()
    for g in gathers(0):
        g.wait()
    store(0, 0).start()

    def chunk(c, _):
        for j in range(2):
            m = 2 + 2 * c + j
            compute_idx(m, j)
            store(j, m).wait()
            for g in gathers(j):
                g.start()
            for g in gathers(1 - j):
                g.wait()
            store(1 - j, m - 1).start()
        return _

    lax.fori_loop(0, (XW - 2) // 2, chunk, None)

    for g in gathers(1):
        g.wait()
    store(1, XW - 1).start()
    store(0, 0).wait()
    store(1, 0).wait()


def kernel(input_len, offsets, pe_table, seg_weight):
    ctbl = _build_combined_table(pe_table, seg_weight)
    off = offsets.astype(jnp.int32)
    ln = input_len.astype(jnp.int32)

    sc = functools.partial(
        pl.kernel,
        out_type=jax.ShapeDtypeStruct((HIST * BATCH, D), jnp.float32),
        mesh=plsc.VectorSubcoreMesh(core_axis_name="c", subcore_axis_name="s"),
        scratch_types=[
            pltpu.VMEM((BW,), jnp.int32),
            pltpu.VMEM((BW,), jnp.int32),
            pltpu.VMEM_SHARED((CROWS, D), jnp.float32),
        ] + [pltpu.VMEM((G,), jnp.int32)] * 4
          + [pltpu.VMEM((BW, D), jnp.float32)] * 2
          + [pltpu.SemaphoreType.DMA] * 6,
    )(_sc_body)
    flat = sc(off, ln, ctbl)
    return flat.reshape(HIST, BATCH, D)


# speedup vs baseline: 1.0705x; 1.0705x over previous
import functools

import jax
import jax.numpy as jnp
from jax import lax
from jax.experimental import pallas as pl
from jax.experimental.pallas import tpu as pltpu
from jax.experimental.pallas import tpu_sc as plsc

D = 128
HIST = 200
BATCH = 4096
SEG_STRIDE = 201
PAD0 = 2 * SEG_STRIDE
PAD1 = PAD0 + 32
CROWS = 512
NW = 32
BW = BATCH // NW
NBUF = 4
L = 16


def _build_tbl_body(pe_ref, sw_ref, out_ref):
    pe = pe_ref[0:SEG_STRIDE]
    sw0 = sw_ref[0:1]
    sw1 = sw_ref[1:2]
    out_ref[0:SEG_STRIDE] = pe + sw0
    out_ref[SEG_STRIDE:PAD0] = pe + sw1
    zeros = jnp.zeros((CROWS - PAD0, D), jnp.float32)
    out_ref[PAD0:PAD1] = zeros[: PAD1 - PAD0] + sw0
    out_ref[PAD1:CROWS] = zeros[: CROWS - PAD1] + sw1


def _build_combined_table(pe_table, seg_weight):
    return pl.pallas_call(
        _build_tbl_body,
        out_shape=jax.ShapeDtypeStruct((CROWS, D), jnp.float32),
    )(pe_table[0:SEG_STRIDE], seg_weight)


def _sc_body(off_hbm, len_hbm, c_hbm, out_hbm,
             off_v, len_v, tbl_sh,
             idx0, idx1, idx2, idx3, rows0, rows1, rows2, rows3,
             gsem0, gsem1, gsem2, gsem3, ssem0, ssem1, ssem2, ssem3):
    cid = lax.axis_index("c")
    sid = lax.axis_index("s")
    wid = sid * 2 + cid
    b0 = wid * BW

    idx = (idx0, idx1, idx2, idx3)
    rows = (rows0, rows1, rows2, rows3)
    gsem = (gsem0, gsem1, gsem2, gsem3)
    ssem = (ssem0, ssem1, ssem2, ssem3)

    tchunk = CROWS // 16
    tsl = pl.ds(sid * tchunk, tchunk)
    pltpu.sync_copy(c_hbm.at[tsl], rows0.at[pl.ds(0, tchunk)])
    pltpu.sync_copy(rows0.at[pl.ds(0, tchunk)], tbl_sh.at[tsl])
    plsc.subcore_barrier()

    pltpu.sync_copy(off_hbm.at[pl.ds(b0, BW)], off_v)
    pltpu.sync_copy(len_hbm.at[pl.ds(b0, BW)], len_v)

    def compute_idx(x, j):
        for s in range(BW // L):
            sl = pl.ds(s * L, L)
            pos = off_v[sl]
            ln = len_v[sl]
            lt = x < pos
            rel = jnp.where(lt, pos - x, x + 1 - pos)
            live = rel + jnp.where(lt, 0, SEG_STRIDE)
            pad = PAD0 + wid + jnp.where(lt, 0, 32)
            idx[j][sl] = jnp.where(x < ln, live, pad)

    def gather(j):
        return pltpu.make_async_copy(tbl_sh.at[idx[j]], rows[j], gsem[j])

    def store(j, x):
        dst = out_hbm.at[pl.ds(x * BATCH + b0, BW)]
        return pltpu.make_async_copy(rows[j], dst, ssem[j])

    for x in range(NBUF):
        compute_idx(x, x)
        gather(x).start()
        if x >= 2:
            gather(x - 2).wait()
            store(x - 2, x - 2).start()

    def chunk(k, _):
        for j in range(NBUF):
            x = NBUF + NBUF * k + j
            compute_idx(x, j)
            store(j, x).wait()
            gather(j).start()
            gather((j - 2) % NBUF).wait()
            store((j - 2) % NBUF, x - 2).start()
        return _

    lax.fori_loop(0, (HIST - NBUF) // NBUF, chunk, None)

    for x in range(HIST - 2, HIST):
        gather(x % NBUF).wait()
        store(x % NBUF, x).start()
    for j in range(NBUF):
        store(j, 0).wait()


def kernel(input_len, offsets, pe_table, seg_weight):
    ctbl = _build_combined_table(pe_table, seg_weight)
    off = offsets.astype(jnp.int32)
    ln = input_len.astype(jnp.int32)

    sc = functools.partial(
        pl.kernel,
        out_type=jax.ShapeDtypeStruct((HIST * BATCH, D), jnp.float32),
        mesh=plsc.VectorSubcoreMesh(core_axis_name="c", subcore_axis_name="s"),
        scratch_types=[
            pltpu.VMEM((BW,), jnp.int32),
            pltpu.VMEM((BW,), jnp.int32),
            pltpu.VMEM_SHARED((CROWS, D), jnp.float32),
        ] + [pltpu.VMEM((BW,), jnp.int32)] * NBUF
          + [pltpu.VMEM((BW, D), jnp.float32)] * NBUF
          + [pltpu.SemaphoreType.DMA] * (2 * NBUF),
    )(_sc_body)
    flat = sc(off, ln, ctbl)
    return flat.reshape(HIST, BATCH, D)
